# baseline (speedup 1.0000x reference)
import functools

import jax
import jax.numpy as jnp
from jax import lax
from jax.experimental import pallas as pl
from jax.experimental.pallas import tpu as pltpu
from jax.experimental.pallas import tpu_sc as plsc

N = 10000
E = 320000
NG = 64
D_IN = 128
D_H = 256
D_OUT = 128

RB = 632
NP = 16 * RB
GRID = NP // RB

CH = 128
NCH = 160
G = 16
NGRP = NCH // G
NCHD = 80
EP = 16 * NCH * CH


def _tc_first_body(x_ref, w_ref, dega_ref, degb_ref, ga_ref, gb_ref, dinv_ref):
    deg = dega_ref[...] + degb_ref[...] - 1.0
    dinvf = lax.rsqrt(jnp.maximum(deg, 1.0))
    dinv_ref[...] = dinvf[:, :16]
    dinv = dinvf[:, 0:1]
    g = dinv * jnp.dot(x_ref[...], w_ref[...], preferred_element_type=jnp.float32)
    ga_ref[...] = g[:, :D_H // 2]
    gb_ref[...] = g[:, D_H // 2:]


def _tc_first(x_p, W1, d0, d1):
    return pl.pallas_call(
        _tc_first_body,
        grid=(GRID,),
        in_specs=[
            pl.BlockSpec((RB, D_IN), lambda i: (i, 0)),
            pl.BlockSpec((D_IN, D_H), lambda i: (0, 0)),
            pl.BlockSpec((RB, 128), lambda i: (i, 0)),
            pl.BlockSpec((RB, 128), lambda i: (i, 0)),
        ],
        out_specs=[
            pl.BlockSpec((RB, D_H // 2), lambda i: (i, 0)),
            pl.BlockSpec((RB, D_H // 2), lambda i: (i, 0)),
            pl.BlockSpec((RB, 16), lambda i: (i, 0)),
        ],
        out_shape=[
            jax.ShapeDtypeStruct((NP, D_H // 2), jnp.float32),
            jax.ShapeDtypeStruct((NP, D_H // 2), jnp.float32),
            jax.ShapeDtypeStruct((NP, 16), jnp.float32),
        ],
    )(x_p, W1, d0, d1)


def _tc_mid_body(ya_ref, yb_ref, w_ref, dinv_ref, b_ref, ga_ref, gb_ref, *, dh):
    dinv = dinv_ref[:, 0:1]
    ha = jnp.maximum(dinv * ya_ref[...] + b_ref[0:1, :D_H // 2], 0.0)
    hb = jnp.maximum(dinv * yb_ref[...] + b_ref[0:1, D_H // 2:], 0.0)
    w = w_ref[...]
    g = dinv * (jnp.dot(ha, w[:D_H // 2], preferred_element_type=jnp.float32)
                + jnp.dot(hb, w[D_H // 2:], preferred_element_type=jnp.float32))
    ga_ref[...] = g[:, :dh // 2]
    gb_ref[...] = g[:, dh // 2:]


def _tc_mid_body_full(ya_ref, yb_ref, w_ref, dinv_ref, b_ref, g_ref):
    dinv = dinv_ref[:, 0:1]
    ha = jnp.maximum(dinv * ya_ref[...] + b_ref[0:1, :D_H // 2], 0.0)
    hb = jnp.maximum(dinv * yb_ref[...] + b_ref[0:1, D_H // 2:], 0.0)
    w = w_ref[...]
    g_ref[...] = dinv * (
        jnp.dot(ha, w[:D_H // 2], preferred_element_type=jnp.float32)
        + jnp.dot(hb, w[D_H // 2:], preferred_element_type=jnp.float32))


def _tc_mid(ya, yb, W, dinv16, b, dh, split=True):
    if split:
        body = functools.partial(_tc_mid_body, dh=dh)
        out_specs = [
            pl.BlockSpec((RB, dh // 2), lambda i: (i, 0)),
            pl.BlockSpec((RB, dh // 2), lambda i: (i, 0)),
        ]
        out_shape = [
            jax.ShapeDtypeStruct((NP, dh // 2), jnp.float32),
            jax.ShapeDtypeStruct((NP, dh // 2), jnp.float32),
        ]
    else:
        body = _tc_mid_body_full
        out_specs = pl.BlockSpec((RB, dh), lambda i: (i, 0))
        out_shape = jax.ShapeDtypeStruct((NP, dh), jnp.float32)
    return pl.pallas_call(
        body,
        grid=(GRID,),
        in_specs=[
            pl.BlockSpec((RB, D_H // 2), lambda i: (i, 0)),
            pl.BlockSpec((RB, D_H // 2), lambda i: (i, 0)),
            pl.BlockSpec((D_H, dh), lambda i: (0, 0)),
            pl.BlockSpec((RB, 16), lambda i: (i, 0)),
            pl.BlockSpec((1, D_H), lambda i: (0, 0)),
        ],
        out_specs=out_specs,
        out_shape=out_shape,
    )(ya, yb, W, dinv16, b.reshape(1, D_H))


def _tc_final_body(p0_ref, p1_ref, g3_ref, dinv_ref, b_ref, batch_ref, wl_ref,
                   bl_ref, out_ref, sums_ref, cnts_ref):
    i = pl.program_id(0)
    dinv = dinv_ref[:, 0:1]
    h = (p0_ref[...] + p1_ref[...] - g3_ref[...]) * dinv + b_ref[0:1, :]
    bm = batch_ref[0, 0, :]
    seg = lax.broadcasted_iota(jnp.int32, (NG, RB), 0)
    oh = (jnp.broadcast_to(bm[None, :], (NG, RB)) == seg).astype(jnp.float32)
    psum = jnp.dot(oh, h, preferred_element_type=jnp.float32)
    pcnt = jnp.sum(oh, axis=1, keepdims=True)

    @pl.when(i == 0)
    def _():
        sums_ref[...] = psum
        cnts_ref[...] = pcnt

    @pl.when(i > 0)
    def _():
        sums_ref[...] += psum
        cnts_ref[...] += pcnt

    @pl.when(i == GRID - 1)
    def _():
        pooled = sums_ref[...] / jnp.maximum(cnts_ref[...], 1.0)
        out_ref[...] = (jnp.dot(pooled, wl_ref[...], preferred_element_type=jnp.float32)
                        + bl_ref[...])


def _tc_final(p0, p1, g3, dinv16, b3, batch_r, Wlin_p, blin_p):
    return pl.pallas_call(
        _tc_final_body,
        grid=(GRID,),
        in_specs=[
            pl.BlockSpec((RB, D_OUT), lambda i: (i, 0)),
            pl.BlockSpec((RB, D_OUT), lambda i: (i, 0)),
            pl.BlockSpec((RB, D_OUT), lambda i: (i, 0)),
            pl.BlockSpec((RB, 16), lambda i: (i, 0)),
            pl.BlockSpec((1, D_OUT), lambda i: (0, 0)),
            pl.BlockSpec((1, 1, RB), lambda i: (i, 0, 0)),
            pl.BlockSpec((D_OUT, 128), lambda i: (0, 0)),
            pl.BlockSpec((1, 128), lambda i: (0, 0)),
        ],
        out_specs=pl.BlockSpec((NG, 128), lambda i: (0, 0)),
        out_shape=jax.ShapeDtypeStruct((NG, 128), jnp.float32),
        scratch_shapes=[
            pltpu.VMEM((NG, 128), jnp.float32),
            pltpu.VMEM((NG, 1), jnp.float32),
        ],
    )(p0, p1, g3, dinv16, b3.reshape(1, D_OUT), batch_r, Wlin_p, blin_p)


_MESH = plsc.VectorSubcoreMesh(core_axis_name="c", subcore_axis_name="s")


def _sc_deg(dstd, ones128):
    @functools.partial(
        pl.kernel, mesh=_MESH,
        out_type=[jax.ShapeDtypeStruct((NP, 128), jnp.float32),
                  jax.ShapeDtypeStruct((NP, 128), jnp.float32)],
        scratch_types=[
            pltpu.VMEM_SHARED((NP, 128), jnp.float32),
            pltpu.VMEM((NCHD, CH), jnp.int32),
            pltpu.VMEM((CH, 128), jnp.float32),
            pltpu.SemaphoreType.DMA,
            pltpu.SemaphoreType.DMA,
        ],
    )
    def k(dst_hbm, ones_h, o0_hbm, o1_hbm, acc_sh, idx_v, ones_v, sem0, sem1):
        c = lax.axis_index("c")
        s = lax.axis_index("s")
        w = c * 16 + s
        base = s * RB
        pltpu.sync_copy(dst_hbm.at[w], idx_v)
        pltpu.sync_copy(ones_h, ones_v)
        for r in range(4):
            pltpu.sync_copy(ones_v, acc_sh.at[pl.ds(base + r * CH, CH)])
        pltpu.sync_copy(ones_v.at[pl.ds(0, 120)],
                        acc_sh.at[pl.ds(base + 4 * CH, 120)])
        plsc.subcore_barrier()

        def body(j, _):
            hs0 = pltpu.async_copy(
                ones_v, acc_sh.at[idx_v.at[2 * j]], sem0, add=True)
            hs1 = pltpu.async_copy(
                ones_v, acc_sh.at[idx_v.at[2 * j + 1]], sem1, add=True)
            hs0.wait()
            hs1.wait()
            return 0

        lax.fori_loop(0, NCHD // 2, body, 0)
        plsc.subcore_barrier()

        @pl.when(c == 0)
        def _():
            pltpu.sync_copy(acc_sh.at[pl.ds(base, RB)], o0_hbm.at[pl.ds(base, RB)])

        @pl.when(c == 1)
        def _():
            pltpu.sync_copy(acc_sh.at[pl.ds(base, RB)], o1_hbm.at[pl.ds(base, RB)])

    return k(dstd, ones128)


def _tile_spmm(g_h, y_h, src_h, dst_h, w, s, ngrp,
               acc_sh, idxs_v, idxd_v, stage, semg, sems0, sems1, semi):
    base = s * RB
    pltpu.sync_copy(src_h.at[w, pl.ds(0, G)], idxs_v.at[0])
    pltpu.sync_copy(dst_h.at[w, pl.ds(0, G)], idxd_v.at[0])
    pltpu.sync_copy(g_h.at[pl.ds(base, RB)], acc_sh.at[pl.ds(base, RB)])
    plsc.subcore_barrier()
    pltpu.async_copy(g_h.at[idxs_v.at[0, 0]], stage.at[0], semg)

    def group(g, _):
        gb = lax.rem(g, 2)
        ngb = 1 - gb

        @pl.when(g < ngrp - 1)
        def _():
            pltpu.async_copy(
                src_h.at[w, pl.ds((g + 1) * G, G)], idxs_v.at[ngb], semi)
            pltpu.async_copy(
                dst_h.at[w, pl.ds((g + 1) * G, G)], idxd_v.at[ngb], semi)

        def body(j, _):
            l0 = 2 * j
            l1 = l0 + 1
            pltpu.make_async_copy(
                g_h.at[idxs_v.at[gb, l0]], stage.at[0], semg).wait()
            hs0 = pltpu.async_copy(
                stage.at[0], acc_sh.at[idxd_v.at[gb, l0]], sems0, add=True)
            hg1 = pltpu.async_copy(
                g_h.at[idxs_v.at[gb, l1]], stage.at[1], semg)
            hg1.wait()
            hs0.wait()
            hs1 = pltpu.async_copy(
                stage.at[1], acc_sh.at[idxd_v.at[gb, l1]], sems1, add=True)

            @pl.when(j < G // 2 - 1)
            def _():
                pltpu.async_copy(
                    g_h.at[idxs_v.at[gb, l0 + 2]], stage.at[0], semg)

            hs1.wait()
            return 0

        lax.fori_loop(0, G // 2, body, 0)

        @pl.when(g < ngrp - 1)
        def _():
            pltpu.make_async_copy(
                src_h.at[w, pl.ds((g + 1) * G, G)], idxs_v.at[ngb],
                semi).wait()
            pltpu.make_async_copy(
                dst_h.at[w, pl.ds((g + 1) * G, G)], idxd_v.at[ngb],
                semi).wait()
            pltpu.async_copy(
                g_h.at[idxs_v.at[ngb, 0]], stage.at[0], semg)
        return 0

    lax.fori_loop(0, ngrp, group, 0)
    plsc.subcore_barrier()
    pltpu.sync_copy(acc_sh.at[pl.ds(base, RB)], y_h.at[pl.ds(base, RB)])


def _sc_spmm_cols(ga, gb, src16, dst16, h):
    @functools.partial(
        pl.kernel, mesh=_MESH,
        out_type=[jax.ShapeDtypeStruct((NP, h), jnp.float32),
                  jax.ShapeDtypeStruct((NP, h), jnp.float32)],
        scratch_types=[
            pltpu.VMEM_SHARED((NP, h), jnp.float32),
            pltpu.VMEM((2, G, CH), jnp.int32),
            pltpu.VMEM((2, G, CH), jnp.int32),
            pltpu.VMEM((2, CH, h), jnp.float32),
            pltpu.SemaphoreType.DMA,
            pltpu.SemaphoreType.DMA,
            pltpu.SemaphoreType.DMA,
            pltpu.SemaphoreType.DMA,
        ],
    )
    def k(ga_h, gb_h, src_h, dst_h, ya_h, yb_h,
          acc_sh, idxs_v, idxd_v, stage, semg, sems0, sems1, semi):
        c = lax.axis_index("c")
        s = lax.axis_index("s")
        scr = (acc_sh, idxs_v, idxd_v, stage, semg, sems0, sems1, semi)

        @pl.when(c == 0)
        def _():
            _tile_spmm(ga_h, ya_h, src_h, dst_h, s, s, NGRP, *scr)

        @pl.when(c == 1)
        def _():
            _tile_spmm(gb_h, yb_h, src_h, dst_h, s, s, NGRP, *scr)

    return k(ga, gb, src16, dst16)


def _sc_spmm_edges(g3, src32, dst32):
    @functools.partial(
        pl.kernel, mesh=_MESH,
        out_type=[jax.ShapeDtypeStruct((NP, D_OUT), jnp.float32),
                  jax.ShapeDtypeStruct((NP, D_OUT), jnp.float32)],
        scratch_types=[
            pltpu.VMEM_SHARED((NP, D_OUT), jnp.float32),
            pltpu.VMEM((2, G, CH), jnp.int32),
            pltpu.VMEM((2, G, CH), jnp.int32),
            pltpu.VMEM((2, CH, D_OUT), jnp.float32),
            pltpu.SemaphoreType.DMA,
            pltpu.SemaphoreType.DMA,
            pltpu.SemaphoreType.DMA,
            pltpu.SemaphoreType.DMA,
        ],
    )
    def k(g_h, src_h, dst_h, p0_h, p1_h,
          acc_sh, idxs_v, idxd_v, stage, semg, sems0, sems1, semi):
        c = lax.axis_index("c")
        s = lax.axis_index("s")
        w = c * 16 + s
        scr = (acc_sh, idxs_v, idxd_v, stage, semg, sems0, sems1, semi)

        @pl.when(c == 0)
        def _():
            _tile_spmm(g_h, p0_h, src_h, dst_h, w, s, NCHD // G, *scr)

        @pl.when(c == 1)
        def _():
            _tile_spmm(g_h, p1_h, src_h, dst_h, w, s, NCHD // G, *scr)

    return k(g3, src32, dst32)


def kernel(x, edge_index, batch, W1, b1, W2, b2, W3, b3, Wlin, blin):
    perm = jnp.argsort(edge_index[0])
    src = edge_index[0][perm]
    dst = edge_index[1][perm]
    pad = EP - E
    src_p = jnp.concatenate([src, jnp.zeros((pad,), jnp.int32)])
    src16 = src_p.reshape(16, NCH, CH)
    srcd = src_p.reshape(32, NCHD, CH)
    dst_p = jnp.concatenate([dst, jnp.full((pad,), N, jnp.int32)])
    dst16 = dst_p.reshape(16, NCH, CH)
    dstd = dst_p.reshape(32, NCHD, CH)
    ones128 = jnp.ones((CH, 128), jnp.float32)

    x_p = jnp.concatenate([x, jnp.zeros((NP - N, D_IN), jnp.float32)])
    batch_r = jnp.concatenate(
        [batch, jnp.full((NP - N,), NG, jnp.int32)]).reshape(GRID, 1, RB)
    Wlin_p = jnp.concatenate([Wlin, jnp.zeros((D_OUT, 126), jnp.float32)], axis=1)
    blin_p = jnp.concatenate([blin, jnp.zeros((126,), jnp.float32)]).reshape(1, 128)

    d0, d1 = _sc_deg(dstd, ones128)
    g1a, g1b, dinv16 = _tc_first(x_p, W1, d0, d1)
    y1a, y1b = _sc_spmm_cols(g1a, g1b, src16, dst16, D_H // 2)
    g2a, g2b = _tc_mid(y1a, y1b, W2, dinv16, b1, D_H)
    y2a, y2b = _sc_spmm_cols(g2a, g2b, src16, dst16, D_H // 2)
    g3 = _tc_mid(y2a, y2b, W3, dinv16, b2, D_OUT, split=False)
    p0, p1 = _sc_spmm_edges(g3, srcd, dstd)
    out = _tc_final(p0, p1, g3, dinv16, b3, batch_r, Wlin_p, blin_p)
    return out[:, :2]


# speedup vs baseline: 1.6003x; 1.6003x over previous
import functools

import jax
import jax.numpy as jnp
from jax import lax
from jax.experimental import pallas as pl
from jax.experimental.pallas import tpu as pltpu
from jax.experimental.pallas import tpu_sc as plsc

N = 10000
E = 320000
NG = 64
D_IN = 128
D_H = 256
D_OUT = 128

RB = 632
NP = 16 * RB
GRID = NP // RB

CH = 64
NCHG = 16
NCH_C = 320
NCH_E = 160
DCH = 128
NCHD = 80
EP = 16 * NCH_C * CH


def _tc_first_body(x_ref, w_ref, dega_ref, degb_ref, ga_ref, gb_ref, dinv_ref):
    deg = dega_ref[...] + degb_ref[...] - 1.0
    dinvf = lax.rsqrt(jnp.maximum(deg, 1.0))
    dinv_ref[...] = dinvf[:, :16]
    dinv = dinvf[:, 0:1]
    g = dinv * jnp.dot(x_ref[...], w_ref[...], preferred_element_type=jnp.float32)
    ga_ref[...] = g[:, :D_H // 2]
    gb_ref[...] = g[:, D_H // 2:]


def _tc_first(x_p, W1, d0, d1):
    return pl.pallas_call(
        _tc_first_body,
        grid=(GRID,),
        in_specs=[
            pl.BlockSpec((RB, D_IN), lambda i: (i, 0)),
            pl.BlockSpec((D_IN, D_H), lambda i: (0, 0)),
            pl.BlockSpec((RB, 128), lambda i: (i, 0)),
            pl.BlockSpec((RB, 128), lambda i: (i, 0)),
        ],
        out_specs=[
            pl.BlockSpec((RB, D_H // 2), lambda i: (i, 0)),
            pl.BlockSpec((RB, D_H // 2), lambda i: (i, 0)),
            pl.BlockSpec((RB, 16), lambda i: (i, 0)),
        ],
        out_shape=[
            jax.ShapeDtypeStruct((NP, D_H // 2), jnp.float32),
            jax.ShapeDtypeStruct((NP, D_H // 2), jnp.float32),
            jax.ShapeDtypeStruct((NP, 16), jnp.float32),
        ],
    )(x_p, W1, d0, d1)


def _tc_mid_body(ya_ref, yb_ref, w_ref, dinv_ref, b_ref, ga_ref, gb_ref, *, dh):
    dinv = dinv_ref[:, 0:1]
    ha = jnp.maximum(dinv * ya_ref[...] + b_ref[0:1, :D_H // 2], 0.0)
    hb = jnp.maximum(dinv * yb_ref[...] + b_ref[0:1, D_H // 2:], 0.0)
    w = w_ref[...]
    g = dinv * (jnp.dot(ha, w[:D_H // 2], preferred_element_type=jnp.float32)
                + jnp.dot(hb, w[D_H // 2:], preferred_element_type=jnp.float32))
    ga_ref[...] = g[:, :dh // 2]
    gb_ref[...] = g[:, dh // 2:]


def _tc_mid_body_full(ya_ref, yb_ref, w_ref, dinv_ref, b_ref, g_ref):
    dinv = dinv_ref[:, 0:1]
    ha = jnp.maximum(dinv * ya_ref[...] + b_ref[0:1, :D_H // 2], 0.0)
    hb = jnp.maximum(dinv * yb_ref[...] + b_ref[0:1, D_H // 2:], 0.0)
    w = w_ref[...]
    g_ref[...] = dinv * (
        jnp.dot(ha, w[:D_H // 2], preferred_element_type=jnp.float32)
        + jnp.dot(hb, w[D_H // 2:], preferred_element_type=jnp.float32))


def _tc_mid(ya, yb, W, dinv16, b, dh, split=True):
    if split:
        body = functools.partial(_tc_mid_body, dh=dh)
        out_specs = [
            pl.BlockSpec((RB, dh // 2), lambda i: (i, 0)),
            pl.BlockSpec((RB, dh // 2), lambda i: (i, 0)),
        ]
        out_shape = [
            jax.ShapeDtypeStruct((NP, dh // 2), jnp.float32),
            jax.ShapeDtypeStruct((NP, dh // 2), jnp.float32),
        ]
    else:
        body = _tc_mid_body_full
        out_specs = pl.BlockSpec((RB, dh), lambda i: (i, 0))
        out_shape = jax.ShapeDtypeStruct((NP, dh), jnp.float32)
    return pl.pallas_call(
        body,
        grid=(GRID,),
        in_specs=[
            pl.BlockSpec((RB, D_H // 2), lambda i: (i, 0)),
            pl.BlockSpec((RB, D_H // 2), lambda i: (i, 0)),
            pl.BlockSpec((D_H, dh), lambda i: (0, 0)),
            pl.BlockSpec((RB, 16), lambda i: (i, 0)),
            pl.BlockSpec((1, D_H), lambda i: (0, 0)),
        ],
        out_specs=out_specs,
        out_shape=out_shape,
    )(ya, yb, W, dinv16, b.reshape(1, D_H))


def _tc_final_body(p0_ref, p1_ref, g3_ref, dinv_ref, b_ref, batch_ref, wl_ref,
                   bl_ref, out_ref, sums_ref, cnts_ref):
    i = pl.program_id(0)
    dinv = dinv_ref[:, 0:1]
    h = (p0_ref[...] + p1_ref[...] - g3_ref[...]) * dinv + b_ref[0:1, :]
    bm = batch_ref[0, 0, :]
    seg = lax.broadcasted_iota(jnp.int32, (NG, RB), 0)
    oh = (jnp.broadcast_to(bm[None, :], (NG, RB)) == seg).astype(jnp.float32)
    psum = jnp.dot(oh, h, preferred_element_type=jnp.float32)
    pcnt = jnp.sum(oh, axis=1, keepdims=True)

    @pl.when(i == 0)
    def _():
        sums_ref[...] = psum
        cnts_ref[...] = pcnt

    @pl.when(i > 0)
    def _():
        sums_ref[...] += psum
        cnts_ref[...] += pcnt

    @pl.when(i == GRID - 1)
    def _():
        pooled = sums_ref[...] / jnp.maximum(cnts_ref[...], 1.0)
        out_ref[...] = (jnp.dot(pooled, wl_ref[...], preferred_element_type=jnp.float32)
                        + bl_ref[...])


def _tc_final(p0, p1, g3, dinv16, b3, batch_r, Wlin_p, blin_p):
    return pl.pallas_call(
        _tc_final_body,
        grid=(GRID,),
        in_specs=[
            pl.BlockSpec((RB, D_OUT), lambda i: (i, 0)),
            pl.BlockSpec((RB, D_OUT), lambda i: (i, 0)),
            pl.BlockSpec((RB, D_OUT), lambda i: (i, 0)),
            pl.BlockSpec((RB, 16), lambda i: (i, 0)),
            pl.BlockSpec((1, D_OUT), lambda i: (0, 0)),
            pl.BlockSpec((1, 1, RB), lambda i: (i, 0, 0)),
            pl.BlockSpec((D_OUT, 128), lambda i: (0, 0)),
            pl.BlockSpec((1, 128), lambda i: (0, 0)),
        ],
        out_specs=pl.BlockSpec((NG, 128), lambda i: (0, 0)),
        out_shape=jax.ShapeDtypeStruct((NG, 128), jnp.float32),
        scratch_shapes=[
            pltpu.VMEM((NG, 128), jnp.float32),
            pltpu.VMEM((NG, 1), jnp.float32),
        ],
    )(p0, p1, g3, dinv16, b3.reshape(1, D_OUT), batch_r, Wlin_p, blin_p)


_MESH = plsc.VectorSubcoreMesh(core_axis_name="c", subcore_axis_name="s")


def _sc_deg(dstd, ones128):
    @functools.partial(
        pl.kernel, mesh=_MESH,
        out_type=[jax.ShapeDtypeStruct((NP, 128), jnp.float32),
                  jax.ShapeDtypeStruct((NP, 128), jnp.float32)],
        scratch_types=[
            pltpu.VMEM_SHARED((NP, 128), jnp.float32),
            pltpu.VMEM((NCHD, DCH), jnp.int32),
            pltpu.VMEM((DCH, 128), jnp.float32),
            pltpu.SemaphoreType.DMA,
            pltpu.SemaphoreType.DMA,
        ],
    )
    def k(dst_hbm, ones_h, o0_hbm, o1_hbm, acc_sh, idx_v, ones_v, sem0, sem1):
        c = lax.axis_index("c")
        s = lax.axis_index("s")
        w = c * 16 + s
        base = s * RB
        pltpu.sync_copy(dst_hbm.at[w], idx_v)
        pltpu.sync_copy(ones_h, ones_v)
        for r in range(4):
            pltpu.sync_copy(ones_v, acc_sh.at[pl.ds(base + r * DCH, DCH)])
        pltpu.sync_copy(ones_v.at[pl.ds(0, 120)],
                        acc_sh.at[pl.ds(base + 4 * DCH, 120)])
        plsc.subcore_barrier()

        def body(j, _):
            hs0 = pltpu.async_copy(
                ones_v, acc_sh.at[idx_v.at[2 * j]], sem0, add=True)
            hs1 = pltpu.async_copy(
                ones_v, acc_sh.at[idx_v.at[2 * j + 1]], sem1, add=True)
            hs0.wait()
            hs1.wait()
            return 0

        lax.fori_loop(0, NCHD // 2, body, 0)
        plsc.subcore_barrier()

        @pl.when(c == 0)
        def _():
            pltpu.sync_copy(acc_sh.at[pl.ds(base, RB)], o0_hbm.at[pl.ds(base, RB)])

        @pl.when(c == 1)
        def _():
            pltpu.sync_copy(acc_sh.at[pl.ds(base, RB)], o1_hbm.at[pl.ds(base, RB)])

    return k(dstd, ones128)


def _tile_spmm(g_h, y_h, src_h, dst_h, w, s, ngrp,
               acc_sh, idxs_v, idxd_v, stage, semg, sems, semi0, semi1):
    base = s * RB
    pltpu.sync_copy(src_h.at[w, pl.ds(0, NCHG)], idxs_v.at[0])
    pltpu.sync_copy(dst_h.at[w, pl.ds(0, NCHG)], idxd_v.at[0])
    pltpu.sync_copy(g_h.at[pl.ds(base, RB)], acc_sh.at[pl.ds(base, RB)])
    plsc.subcore_barrier()
    pltpu.async_copy(g_h.at[idxs_v.at[0, 0]], stage.at[0], semg[0])
    pltpu.async_copy(g_h.at[idxs_v.at[0, 1]], stage.at[1], semg[1])

    def group(g, _):
        gb = lax.rem(g, 2)
        ngb = 1 - gb

        @pl.when(g < ngrp - 1)
        def _():
            pltpu.async_copy(
                src_h.at[w, pl.ds((g + 1) * NCHG, NCHG)], idxs_v.at[ngb],
                semi0)
            pltpu.async_copy(
                dst_h.at[w, pl.ds((g + 1) * NCHG, NCHG)], idxd_v.at[ngb],
                semi1)

        for l in range(NCHG):
            b = l % 4
            b2 = (l + 2) % 4

            def wait_scatter(bb, row):
                pltpu.make_async_copy(
                    stage.at[bb], acc_sh.at[idxd_v.at[gb, row]],
                    sems[bb]).wait()

            pltpu.make_async_copy(
                g_h.at[idxs_v.at[gb, l]], stage.at[b], semg[b]).wait()
            pltpu.async_copy(
                stage.at[b], acc_sh.at[idxd_v.at[gb, l]], sems[b], add=True)

            if l < 2:
                @pl.when(g > 0)
                def _(b2=b2, l=l):
                    wait_scatter(b2, l)
                pltpu.async_copy(
                    g_h.at[idxs_v.at[gb, l + 2]], stage.at[b2], semg[b2])
            elif l < NCHG - 2:
                wait_scatter(b2, l)
                pltpu.async_copy(
                    g_h.at[idxs_v.at[gb, l + 2]], stage.at[b2], semg[b2])
            else:
                if l == NCHG - 2:
                    @pl.when(g < ngrp - 1)
                    def _():
                        pltpu.make_async_copy(
                            src_h.at[w, pl.ds((g + 1) * NCHG, NCHG)],
                            idxs_v.at[ngb], semi0).wait()
                        pltpu.make_async_copy(
                            dst_h.at[w, pl.ds((g + 1) * NCHG, NCHG)],
                            idxd_v.at[ngb], semi1).wait()
                wait_scatter(b2, l)

                @pl.when(g < ngrp - 1)
                def _(b2=b2, l=l):
                    pltpu.async_copy(
                        g_h.at[idxs_v.at[ngb, l + 2 - NCHG]], stage.at[b2],
                        semg[b2])
        return 0

    lax.fori_loop(0, ngrp, group, 0)
    pltpu.make_async_copy(
        stage.at[(NCHG - 2) % 4], acc_sh.at[idxd_v.at[0, 0]],
        sems[(NCHG - 2) % 4]).wait()
    pltpu.make_async_copy(
        stage.at[(NCHG - 1) % 4], acc_sh.at[idxd_v.at[0, 1]],
        sems[(NCHG - 1) % 4]).wait()
    plsc.subcore_barrier()
    pltpu.sync_copy(acc_sh.at[pl.ds(base, RB)], y_h.at[pl.ds(base, RB)])


_SPMM_SCRATCH = [
    pltpu.VMEM((2, NCHG, CH), jnp.int32),
    pltpu.VMEM((2, NCHG, CH), jnp.int32),
] + [pltpu.SemaphoreType.DMA] * 10


def _sc_spmm_cols(ga, gb, src16, dst16, h):
    @functools.partial(
        pl.kernel, mesh=_MESH,
        out_type=[jax.ShapeDtypeStruct((NP, h), jnp.float32),
                  jax.ShapeDtypeStruct((NP, h), jnp.float32)],
        scratch_types=[pltpu.VMEM_SHARED((NP, h), jnp.float32),
                       pltpu.VMEM((4, CH, h), jnp.float32)] + _SPMM_SCRATCH,
    )
    def k(ga_h, gb_h, src_h, dst_h, ya_h, yb_h,
          acc_sh, stage, idxs_v, idxd_v,
          sg0, sg1, sg2, sg3, ss0, ss1, ss2, ss3, semi0, semi1):
        c = lax.axis_index("c")
        s = lax.axis_index("s")
        scr = (acc_sh, idxs_v, idxd_v, stage,
               (sg0, sg1, sg2, sg3), (ss0, ss1, ss2, ss3), semi0, semi1)

        @pl.when(c == 0)
        def _():
            _tile_spmm(ga_h, ya_h, src_h, dst_h, s, s, NCH_C // NCHG, *scr)

        @pl.when(c == 1)
        def _():
            _tile_spmm(gb_h, yb_h, src_h, dst_h, s, s, NCH_C // NCHG, *scr)

    return k(ga, gb, src16, dst16)


def _sc_spmm_edges(g3, src32, dst32):
    @functools.partial(
        pl.kernel, mesh=_MESH,
        out_type=[jax.ShapeDtypeStruct((NP, D_OUT), jnp.float32),
                  jax.ShapeDtypeStruct((NP, D_OUT), jnp.float32)],
        scratch_types=[pltpu.VMEM_SHARED((NP, D_OUT), jnp.float32),
                       pltpu.VMEM((4, CH, D_OUT), jnp.float32)]
        + _SPMM_SCRATCH,
    )
    def k(g_h, src_h, dst_h, p0_h, p1_h,
          acc_sh, stage, idxs_v, idxd_v,
          sg0, sg1, sg2, sg3, ss0, ss1, ss2, ss3, semi0, semi1):
        c = lax.axis_index("c")
        s = lax.axis_index("s")
        w = c * 16 + s
        scr = (acc_sh, idxs_v, idxd_v, stage,
               (sg0, sg1, sg2, sg3), (ss0, ss1, ss2, ss3), semi0, semi1)

        @pl.when(c == 0)
        def _():
            _tile_spmm(g_h, p0_h, src_h, dst_h, w, s, NCH_E // NCHG, *scr)

        @pl.when(c == 1)
        def _():
            _tile_spmm(g_h, p1_h, src_h, dst_h, w, s, NCH_E // NCHG, *scr)

    return k(g3, src32, dst32)


def kernel(x, edge_index, batch, W1, b1, W2, b2, W3, b3, Wlin, blin):
    src = edge_index[0]
    dst = edge_index[1]
    pad = EP - E
    src_p = jnp.concatenate([src, jnp.zeros((pad,), jnp.int32)])
    src16 = src_p.reshape(16, NCH_C, CH)
    srcd = src_p.reshape(32, NCH_E, CH)
    dst_p = jnp.concatenate([dst, jnp.full((pad,), N, jnp.int32)])
    dst16 = dst_p.reshape(16, NCH_C, CH)
    dstd = dst_p.reshape(32, NCH_E, CH)
    dstdeg = dst_p.reshape(32, NCHD, DCH)
    ones128 = jnp.ones((DCH, 128), jnp.float32)

    x_p = jnp.concatenate([x, jnp.zeros((NP - N, D_IN), jnp.float32)])
    batch_r = jnp.concatenate(
        [batch, jnp.full((NP - N,), NG, jnp.int32)]).reshape(GRID, 1, RB)
    Wlin_p = jnp.concatenate([Wlin, jnp.zeros((D_OUT, 126), jnp.float32)], axis=1)
    blin_p = jnp.concatenate([blin, jnp.zeros((126,), jnp.float32)]).reshape(1, 128)

    d0, d1 = _sc_deg(dstdeg, ones128)
    g1a, g1b, dinv16 = _tc_first(x_p, W1, d0, d1)
    y1a, y1b = _sc_spmm_cols(g1a, g1b, src16, dst16, D_H // 2)
    g2a, g2b = _tc_mid(y1a, y1b, W2, dinv16, b1, D_H)
    y2a, y2b = _sc_spmm_cols(g2a, g2b, src16, dst16, D_H // 2)
    g3 = _tc_mid(y2a, y2b, W3, dinv16, b2, D_OUT, split=False)
    p0, p1 = _sc_spmm_edges(g3, srcd, dstd)
    out = _tc_final(p0, p1, g3, dinv16, b3, batch_r, Wlin_p, blin_p)
    return out[:, :2]


# speedup vs baseline: 1.6565x; 1.0351x over previous
import functools

import jax
import jax.numpy as jnp
from jax import lax
from jax.experimental import pallas as pl
from jax.experimental.pallas import tpu as pltpu
from jax.experimental.pallas import tpu_sc as plsc

N = 10000
E = 320000
NG = 64
D_IN = 128
D_H = 256
D_OUT = 128

RB = 632
NP = 16 * RB
GRID = NP // RB

CH = 64
NCHG = 16
NCH_C = 320
NCH_E = 160
DCH = 128
NCHD = 80
EP = 16 * NCH_C * CH


def _tc_first_body(x_ref, w_ref, dega_ref, degb_ref, ga_ref, gb_ref, dinv_ref):
    deg = dega_ref[...] + degb_ref[...] - 1.0
    dinvf = lax.rsqrt(jnp.maximum(deg, 1.0))
    dinv_ref[...] = dinvf[:, :16]
    dinv = dinvf[:, 0:1]
    g = dinv * jnp.dot(x_ref[...], w_ref[...], preferred_element_type=jnp.float32)
    ga_ref[...] = g[:, :D_H // 2]
    gb_ref[...] = g[:, D_H // 2:]


def _tc_first(x_p, W1, d0, d1):
    return pl.pallas_call(
        _tc_first_body,
        grid=(GRID,),
        in_specs=[
            pl.BlockSpec((RB, D_IN), lambda i: (i, 0)),
            pl.BlockSpec((D_IN, D_H), lambda i: (0, 0)),
            pl.BlockSpec((RB, 128), lambda i: (i, 0)),
            pl.BlockSpec((RB, 128), lambda i: (i, 0)),
        ],
        out_specs=[
            pl.BlockSpec((RB, D_H // 2), lambda i: (i, 0)),
            pl.BlockSpec((RB, D_H // 2), lambda i: (i, 0)),
            pl.BlockSpec((RB, 16), lambda i: (i, 0)),
        ],
        out_shape=[
            jax.ShapeDtypeStruct((NP, D_H // 2), jnp.float32),
            jax.ShapeDtypeStruct((NP, D_H // 2), jnp.float32),
            jax.ShapeDtypeStruct((NP, 16), jnp.float32),
        ],
    )(x_p, W1, d0, d1)


def _tc_mid_body(ya_ref, yb_ref, w_ref, dinv_ref, b_ref, ga_ref, gb_ref, *, dh):
    dinv = dinv_ref[:, 0:1]
    ha = jnp.maximum(dinv * ya_ref[...] + b_ref[0:1, :D_H // 2], 0.0)
    hb = jnp.maximum(dinv * yb_ref[...] + b_ref[0:1, D_H // 2:], 0.0)
    w = w_ref[...]
    g = dinv * (jnp.dot(ha, w[:D_H // 2], preferred_element_type=jnp.float32)
                + jnp.dot(hb, w[D_H // 2:], preferred_element_type=jnp.float32))
    ga_ref[...] = g[:, :dh // 2]
    gb_ref[...] = g[:, dh // 2:]


def _tc_mid_body_full(ya_ref, yb_ref, w_ref, dinv_ref, b_ref, g_ref):
    dinv = dinv_ref[:, 0:1]
    ha = jnp.maximum(dinv * ya_ref[...] + b_ref[0:1, :D_H // 2], 0.0)
    hb = jnp.maximum(dinv * yb_ref[...] + b_ref[0:1, D_H // 2:], 0.0)
    w = w_ref[...]
    g_ref[...] = dinv * (
        jnp.dot(ha, w[:D_H // 2], preferred_element_type=jnp.float32)
        + jnp.dot(hb, w[D_H // 2:], preferred_element_type=jnp.float32))


def _tc_mid(ya, yb, W, dinv16, b, dh, split=True):
    if split:
        body = functools.partial(_tc_mid_body, dh=dh)
        out_specs = [
            pl.BlockSpec((RB, dh // 2), lambda i: (i, 0)),
            pl.BlockSpec((RB, dh // 2), lambda i: (i, 0)),
        ]
        out_shape = [
            jax.ShapeDtypeStruct((NP, dh // 2), jnp.float32),
            jax.ShapeDtypeStruct((NP, dh // 2), jnp.float32),
        ]
    else:
        body = _tc_mid_body_full
        out_specs = pl.BlockSpec((RB, dh), lambda i: (i, 0))
        out_shape = jax.ShapeDtypeStruct((NP, dh), jnp.float32)
    return pl.pallas_call(
        body,
        grid=(GRID,),
        in_specs=[
            pl.BlockSpec((RB, D_H // 2), lambda i: (i, 0)),
            pl.BlockSpec((RB, D_H // 2), lambda i: (i, 0)),
            pl.BlockSpec((D_H, dh), lambda i: (0, 0)),
            pl.BlockSpec((RB, 16), lambda i: (i, 0)),
            pl.BlockSpec((1, D_H), lambda i: (0, 0)),
        ],
        out_specs=out_specs,
        out_shape=out_shape,
    )(ya, yb, W, dinv16, b.reshape(1, D_H))


def _tc_final_body(p0_ref, p1_ref, g3_ref, dinv_ref, b_ref, batch_ref, wl_ref,
                   bl_ref, out_ref, sums_ref, cnts_ref):
    i = pl.program_id(0)
    dinv = dinv_ref[:, 0:1]
    h = (p0_ref[...] + p1_ref[...] - g3_ref[...]) * dinv + b_ref[0:1, :]
    bm = batch_ref[0, 0, :]
    seg = lax.broadcasted_iota(jnp.int32, (NG, RB), 0)
    oh = (jnp.broadcast_to(bm[None, :], (NG, RB)) == seg).astype(jnp.float32)
    psum = jnp.dot(oh, h, preferred_element_type=jnp.float32)
    pcnt = jnp.sum(oh, axis=1, keepdims=True)

    @pl.when(i == 0)
    def _():
        sums_ref[...] = psum
        cnts_ref[...] = pcnt

    @pl.when(i > 0)
    def _():
        sums_ref[...] += psum
        cnts_ref[...] += pcnt

    @pl.when(i == GRID - 1)
    def _():
        pooled = sums_ref[...] / jnp.maximum(cnts_ref[...], 1.0)
        out_ref[...] = (jnp.dot(pooled, wl_ref[...], preferred_element_type=jnp.float32)
                        + bl_ref[...])


def _tc_final(p0, p1, g3, dinv16, b3, batch_r, Wlin_p, blin_p):
    return pl.pallas_call(
        _tc_final_body,
        grid=(GRID,),
        in_specs=[
            pl.BlockSpec((RB, D_OUT), lambda i: (i, 0)),
            pl.BlockSpec((RB, D_OUT), lambda i: (i, 0)),
            pl.BlockSpec((RB, D_OUT), lambda i: (i, 0)),
            pl.BlockSpec((RB, 16), lambda i: (i, 0)),
            pl.BlockSpec((1, D_OUT), lambda i: (0, 0)),
            pl.BlockSpec((1, 1, RB), lambda i: (i, 0, 0)),
            pl.BlockSpec((D_OUT, 128), lambda i: (0, 0)),
            pl.BlockSpec((1, 128), lambda i: (0, 0)),
        ],
        out_specs=pl.BlockSpec((NG, 128), lambda i: (0, 0)),
        out_shape=jax.ShapeDtypeStruct((NG, 128), jnp.float32),
        scratch_shapes=[
            pltpu.VMEM((NG, 128), jnp.float32),
            pltpu.VMEM((NG, 1), jnp.float32),
        ],
    )(p0, p1, g3, dinv16, b3.reshape(1, D_OUT), batch_r, Wlin_p, blin_p)


_MESH = plsc.VectorSubcoreMesh(core_axis_name="c", subcore_axis_name="s")


def _sc_deg(dstd, ones128):
    @functools.partial(
        pl.kernel, mesh=_MESH,
        out_type=[jax.ShapeDtypeStruct((NP, 128), jnp.float32),
                  jax.ShapeDtypeStruct((NP, 128), jnp.float32)],
        scratch_types=[
            pltpu.VMEM_SHARED((NP, 128), jnp.float32),
            pltpu.VMEM((NCHD, DCH), jnp.int32),
            pltpu.VMEM((DCH, 128), jnp.float32),
            pltpu.SemaphoreType.DMA,
            pltpu.SemaphoreType.DMA,
        ],
    )
    def k(dst_hbm, ones_h, o0_hbm, o1_hbm, acc_sh, idx_v, ones_v, sem0, sem1):
        c = lax.axis_index("c")
        s = lax.axis_index("s")
        w = c * 16 + s
        base = s * RB
        pltpu.sync_copy(dst_hbm.at[w], idx_v)
        pltpu.sync_copy(ones_h, ones_v)
        for r in range(4):
            pltpu.sync_copy(ones_v, acc_sh.at[pl.ds(base + r * DCH, DCH)])
        pltpu.sync_copy(ones_v.at[pl.ds(0, 120)],
                        acc_sh.at[pl.ds(base + 4 * DCH, 120)])
        plsc.subcore_barrier()

        def body(j, _):
            hs0 = pltpu.async_copy(
                ones_v, acc_sh.at[idx_v.at[2 * j]], sem0, add=True)
            hs1 = pltpu.async_copy(
                ones_v, acc_sh.at[idx_v.at[2 * j + 1]], sem1, add=True)
            hs0.wait()
            hs1.wait()
            return 0

        lax.fori_loop(0, NCHD // 2, body, 0)
        plsc.subcore_barrier()

        @pl.when(c == 0)
        def _():
            pltpu.sync_copy(acc_sh.at[pl.ds(base, RB)], o0_hbm.at[pl.ds(base, RB)])

        @pl.when(c == 1)
        def _():
            pltpu.sync_copy(acc_sh.at[pl.ds(base, RB)], o1_hbm.at[pl.ds(base, RB)])

    return k(dstd, ones128)


def _tile_spmm(g_h, y_h, src_h, dst_h, w, s, ngrp,
               acc_sh, idxs_v, idxd_v, stage, semg, sems, semi0, semi1):
    base = s * RB
    pltpu.sync_copy(src_h.at[w, pl.ds(0, NCHG)], idxs_v.at[0])
    pltpu.sync_copy(dst_h.at[w, pl.ds(0, NCHG)], idxd_v.at[0])
    pltpu.sync_copy(g_h.at[pl.ds(base, RB)], acc_sh.at[pl.ds(base, RB)])
    plsc.subcore_barrier()
    pltpu.async_copy(g_h.at[idxs_v.at[0, 0]], stage.at[0], semg[0])
    pltpu.async_copy(g_h.at[idxs_v.at[0, 1]], stage.at[1], semg[1])

    def group(g, _):
        gb = lax.rem(g, 2)
        ngb = 1 - gb

        @pl.when(g < ngrp - 1)
        def _():
            pltpu.async_copy(
                src_h.at[w, pl.ds((g + 1) * NCHG, NCHG)], idxs_v.at[ngb],
                semi0)
            pltpu.async_copy(
                dst_h.at[w, pl.ds((g + 1) * NCHG, NCHG)], idxd_v.at[ngb],
                semi1)

        for l in range(NCHG):
            b = l % 4
            b2 = (l + 2) % 4

            def wait_scatter(bb, row):
                pltpu.make_async_copy(
                    stage.at[bb], acc_sh.at[idxd_v.at[gb, row]],
                    sems[bb]).wait()

            pltpu.make_async_copy(
                g_h.at[idxs_v.at[gb, l]], stage.at[b], semg[b]).wait()
            pltpu.async_copy(
                stage.at[b], acc_sh.at[idxd_v.at[gb, l]], sems[b], add=True)

            if l < 2:
                @pl.when(g > 0)
                def _(b2=b2, l=l):
                    wait_scatter(b2, l)
                pltpu.async_copy(
                    g_h.at[idxs_v.at[gb, l + 2]], stage.at[b2], semg[b2])
            elif l < NCHG - 2:
                wait_scatter(b2, l)
                pltpu.async_copy(
                    g_h.at[idxs_v.at[gb, l + 2]], stage.at[b2], semg[b2])
            else:
                if l == NCHG - 2:
                    @pl.when(g < ngrp - 1)
                    def _():
                        pltpu.make_async_copy(
                            src_h.at[w, pl.ds((g + 1) * NCHG, NCHG)],
                            idxs_v.at[ngb], semi0).wait()
                        pltpu.make_async_copy(
                            dst_h.at[w, pl.ds((g + 1) * NCHG, NCHG)],
                            idxd_v.at[ngb], semi1).wait()
                wait_scatter(b2, l)

                @pl.when(g < ngrp - 1)
                def _(b2=b2, l=l):
                    pltpu.async_copy(
                        g_h.at[idxs_v.at[ngb, l + 2 - NCHG]], stage.at[b2],
                        semg[b2])
        return 0

    lax.fori_loop(0, ngrp, group, 0)
    pltpu.make_async_copy(
        stage.at[(NCHG - 2) % 4], acc_sh.at[idxd_v.at[0, 0]],
        sems[(NCHG - 2) % 4]).wait()
    pltpu.make_async_copy(
        stage.at[(NCHG - 1) % 4], acc_sh.at[idxd_v.at[0, 1]],
        sems[(NCHG - 1) % 4]).wait()
    plsc.subcore_barrier()
    pltpu.sync_copy(acc_sh.at[pl.ds(base, RB)], y_h.at[pl.ds(base, RB)])


_SPMM_SCRATCH = [
    pltpu.VMEM((2, NCHG, CH), jnp.int32),
    pltpu.VMEM((2, NCHG, CH), jnp.int32),
] + [pltpu.SemaphoreType.DMA] * 10


def _sc_spmm_cols(ga, gb, src16, dst16, h):
    @functools.partial(
        pl.kernel, mesh=_MESH,
        out_type=[jax.ShapeDtypeStruct((NP, h), jnp.float32),
                  jax.ShapeDtypeStruct((NP, h), jnp.float32)],
        scratch_types=[pltpu.VMEM_SHARED((NP, h), jnp.float32),
                       pltpu.VMEM((4, CH, h), jnp.float32)] + _SPMM_SCRATCH,
    )
    def k(ga_h, gb_h, src_h, dst_h, ya_h, yb_h,
          acc_sh, stage, idxs_v, idxd_v,
          sg0, sg1, sg2, sg3, ss0, ss1, ss2, ss3, semi0, semi1):
        c = lax.axis_index("c")
        s = lax.axis_index("s")
        scr = (acc_sh, idxs_v, idxd_v, stage,
               (sg0, sg1, sg2, sg3), (ss0, ss1, ss2, ss3), semi0, semi1)

        @pl.when(c == 0)
        def _():
            _tile_spmm(ga_h, ya_h, src_h, dst_h, s, s, NCH_C // NCHG, *scr)

        @pl.when(c == 1)
        def _():
            _tile_spmm(gb_h, yb_h, src_h, dst_h, s, s, NCH_C // NCHG, *scr)

    return k(ga, gb, src16, dst16)


def _sc_spmm_edges(g3, src32, dst32):
    @functools.partial(
        pl.kernel, mesh=_MESH,
        out_type=[jax.ShapeDtypeStruct((NP, D_OUT), jnp.float32),
                  jax.ShapeDtypeStruct((NP, D_OUT), jnp.float32)],
        scratch_types=[pltpu.VMEM_SHARED((NP, D_OUT), jnp.float32),
                       pltpu.VMEM((4, CH, D_OUT), jnp.float32)]
        + _SPMM_SCRATCH,
    )
    def k(g_h, src_h, dst_h, p0_h, p1_h,
          acc_sh, stage, idxs_v, idxd_v,
          sg0, sg1, sg2, sg3, ss0, ss1, ss2, ss3, semi0, semi1):
        c = lax.axis_index("c")
        s = lax.axis_index("s")
        w = c * 16 + s
        scr = (acc_sh, idxs_v, idxd_v, stage,
               (sg0, sg1, sg2, sg3), (ss0, ss1, ss2, ss3), semi0, semi1)

        @pl.when(c == 0)
        def _():
            _tile_spmm(g_h, p0_h, src_h, dst_h, w, s, NCH_E // NCHG, *scr)

        @pl.when(c == 1)
        def _():
            _tile_spmm(g_h, p1_h, src_h, dst_h, w, s, NCH_E // NCHG, *scr)

    return k(g3, src32, dst32)


def kernel(x, edge_index, batch, W1, b1, W2, b2, W3, b3, Wlin, blin):
    src = edge_index[0]
    dst = edge_index[1]
    pad = EP - E
    src_p = jnp.concatenate([src, jnp.zeros((pad,), jnp.int32)])
    src16 = src_p.reshape(16, NCH_C, CH)
    srcd = src_p.reshape(32, NCH_E, CH)
    pad_dst = N + (jnp.arange(pad, dtype=jnp.int32) % (NP - N))
    dst_p = jnp.concatenate([dst, pad_dst])
    dst16 = dst_p.reshape(16, NCH_C, CH)
    dstd = dst_p.reshape(32, NCH_E, CH)
    dstdeg = dst_p.reshape(32, NCHD, DCH)
    ones128 = jnp.ones((DCH, 128), jnp.float32)

    x_p = jnp.concatenate([x, jnp.zeros((NP - N, D_IN), jnp.float32)])
    batch_r = jnp.concatenate(
        [batch, jnp.full((NP - N,), NG, jnp.int32)]).reshape(GRID, 1, RB)
    Wlin_p = jnp.concatenate([Wlin, jnp.zeros((D_OUT, 126), jnp.float32)], axis=1)
    blin_p = jnp.concatenate([blin, jnp.zeros((126,), jnp.float32)]).reshape(1, 128)

    d0, d1 = _sc_deg(dstdeg, ones128)
    g1a, g1b, dinv16 = _tc_first(x_p, W1, d0, d1)
    y1a, y1b = _sc_spmm_cols(g1a, g1b, src16, dst16, D_H // 2)
    g2a, g2b = _tc_mid(y1a, y1b, W2, dinv16, b1, D_H)
    y2a, y2b = _sc_spmm_cols(g2a, g2b, src16, dst16, D_H // 2)
    g3 = _tc_mid(y2a, y2b, W3, dinv16, b2, D_OUT, split=False)
    p0, p1 = _sc_spmm_edges(g3, srcd, dstd)
    out = _tc_final(p0, p1, g3, dinv16, b3, batch_r, Wlin_p, blin_p)
    return out[:, :2]
